# baseline (speedup 1.0000x reference)
import functools

import jax
import jax.numpy as jnp
from jax.experimental import pallas as pl
from jax.experimental.pallas import tpu as pltpu


def _lenet_body(x_ref, wb1_ref, b1_ref, wb2_ref, b2_ref,
                wf1_ref, bf1_ref, wf2_ref, bf2_ref,
                wf3_ref, bf3_ref, out_ref, *, tb):
    f32 = jnp.float32
    bf16 = jnp.bfloat16
    R1 = 30 * tb
    R2 = 25 * tb

    x4 = x_ref[...].reshape(tb, 96, 32)
    xt = jnp.transpose(x4, (1, 0, 2))
    x = jnp.concatenate(
        [xt[32 * c:32 * (c + 1)].reshape(32 * tb, 32) for c in range(3)],
        axis=1).astype(bf16)
    acc1 = jnp.dot(x[0:R1], wb1_ref[0], preferred_element_type=f32)
    acc1 = acc1 + jnp.dot(x[tb:tb + R1], wb1_ref[1], preferred_element_type=f32)
    acc1 = acc1 + jnp.dot(x[2 * tb:2 * tb + R1], wb1_ref[2],
                          preferred_element_type=f32)
    y1 = jnp.maximum(acc1 + b1_ref[...], 0.0).astype(bf16)

    hm1 = jnp.maximum(y1[:-tb], y1[tb:])
    wm1 = jnp.maximum(hm1[:, :174], hm1[:, 6:180])

    acc2 = jnp.dot(wm1[0:R2], wb2_ref[0], preferred_element_type=f32)
    acc2 = acc2 + jnp.dot(wm1[2 * tb:2 * tb + R2], wb2_ref[1],
                          preferred_element_type=f32)
    acc2 = acc2 + jnp.dot(wm1[4 * tb:4 * tb + R2], wb2_ref[2],
                          preferred_element_type=f32)
    y2 = jnp.maximum(acc2 + b2_ref[...], 0.0).astype(bf16)

    hm2 = jnp.maximum(y2[:-2 * tb], y2[2 * tb:])
    wm2 = jnp.maximum(hm2[:, :192], hm2[:, 16:208])

    acc_f = jnp.dot(wm2[0:tb], wf1_ref[0], preferred_element_type=f32)
    for h4 in range(1, 6):
        acc_f = acc_f + jnp.dot(wm2[4 * h4 * tb:4 * h4 * tb + tb], wf1_ref[h4],
                                preferred_element_type=f32)
    z1 = jnp.maximum(acc_f + bf1_ref[...], 0.0).astype(bf16)

    z2 = jnp.maximum(jnp.dot(z1, wf2_ref[...], preferred_element_type=f32)
                     + bf2_ref[...], 0.0).astype(bf16)
    z3 = jnp.dot(z2, wf3_ref[...], preferred_element_type=f32) + bf3_ref[...]

    out_ref[...] = z3.astype(out_ref.dtype)


def kernel(x_nchw, wb1, b1, wb2, b2, wf1, bf1, wf2, bf2, wf3, bf3):
    f32 = jnp.float32
    bf16 = jnp.bfloat16
    tb = 128

    B = x_nchw.shape[0]
    Bp = ((B + tb - 1) // tb) * tb
    if Bp != B:
        x_nchw = jnp.pad(x_nchw, ((0, Bp - B), (0, 0), (0, 0), (0, 0)))
    G = Bp // tb

    perm = jnp.arange(96)
    wi, ci = perm % 32, perm // 32
    wb1p = wb1[:, wi * 3 + ci, :]

    body = functools.partial(_lenet_body, tb=tb)
    out = pl.pallas_call(
        body,
        out_shape=jax.ShapeDtypeStruct((Bp, 128), f32),
        grid=(G,),
        in_specs=[
            pl.BlockSpec((tb, 3, 32, 32), lambda i: (i, 0, 0, 0)),
            pl.BlockSpec((3, 96, 180), lambda i: (0, 0, 0)),
            pl.BlockSpec((1, 180), lambda i: (0, 0)),
            pl.BlockSpec((3, 174, 208), lambda i: (0, 0, 0)),
            pl.BlockSpec((1, 208), lambda i: (0, 0)),
            pl.BlockSpec((6, 192, 120), lambda i: (0, 0, 0)),
            pl.BlockSpec((1, 120), lambda i: (0, 0)),
            pl.BlockSpec((120, 84), lambda i: (0, 0)),
            pl.BlockSpec((1, 84), lambda i: (0, 0)),
            pl.BlockSpec((84, 128), lambda i: (0, 0)),
            pl.BlockSpec((1, 128), lambda i: (0, 0)),
        ],
        out_specs=pl.BlockSpec((tb, 128), lambda i: (i, 0)),
        compiler_params=pltpu.CompilerParams(
            dimension_semantics=("parallel",),
            vmem_limit_bytes=64 * 1024 * 1024),
    )(x_nchw, wb1p.astype(bf16), b1, wb2.astype(bf16), b2,
      wf1.astype(bf16), bf1, wf2.astype(bf16), bf2, wf3.astype(bf16), bf3)

    return out[:B, :10]


# speedup vs baseline: 1.2855x; 1.2855x over previous
import functools

import jax
import jax.numpy as jnp
from jax.experimental import pallas as pl
from jax.experimental.pallas import tpu as pltpu


def _lenet_body(x_ref, wb1_ref, b1_ref, wb2_ref, b2_ref,
                wf1_ref, bf1_ref, wf2_ref, bf2_ref,
                wf3_ref, bf3_ref, out_ref, *, tb):
    f32 = jnp.float32
    bf16 = jnp.bfloat16
    R1 = 30 * tb
    R2 = 25 * tb

    x = jnp.transpose(x_ref[...], (1, 0, 2)).reshape(32 * tb, 96).astype(bf16)
    acc1 = jnp.dot(x[0:R1], wb1_ref[0], preferred_element_type=f32)
    acc1 = acc1 + jnp.dot(x[tb:tb + R1], wb1_ref[1], preferred_element_type=f32)
    acc1 = acc1 + jnp.dot(x[2 * tb:2 * tb + R1], wb1_ref[2],
                          preferred_element_type=f32)
    y1 = jnp.maximum(acc1 + b1_ref[...], 0.0).astype(bf16)

    hm1 = jnp.maximum(y1[:-tb], y1[tb:])
    wm1 = jnp.maximum(hm1[:, :174], hm1[:, 6:180])

    acc2 = jnp.dot(wm1[0:R2], wb2_ref[0], preferred_element_type=f32)
    acc2 = acc2 + jnp.dot(wm1[2 * tb:2 * tb + R2], wb2_ref[1],
                          preferred_element_type=f32)
    acc2 = acc2 + jnp.dot(wm1[4 * tb:4 * tb + R2], wb2_ref[2],
                          preferred_element_type=f32)
    y2 = jnp.maximum(acc2 + b2_ref[...], 0.0).astype(bf16)

    hm2 = jnp.maximum(y2[:-2 * tb], y2[2 * tb:])
    wm2 = jnp.maximum(hm2[:, :192], hm2[:, 16:208])

    acc_f = jnp.dot(wm2[0:tb], wf1_ref[0], preferred_element_type=f32)
    for h4 in range(1, 6):
        acc_f = acc_f + jnp.dot(wm2[4 * h4 * tb:4 * h4 * tb + tb], wf1_ref[h4],
                                preferred_element_type=f32)
    z1 = jnp.maximum(acc_f + bf1_ref[...], 0.0).astype(bf16)

    z2 = jnp.maximum(jnp.dot(z1, wf2_ref[...], preferred_element_type=f32)
                     + bf2_ref[...], 0.0).astype(bf16)
    z3 = jnp.dot(z2, wf3_ref[...], preferred_element_type=f32) + bf3_ref[...]

    out_ref[...] = z3.astype(out_ref.dtype)


def kernel(x_nchw, wb1, b1, wb2, b2, wf1, bf1, wf2, bf2, wf3, bf3):
    f32 = jnp.float32
    bf16 = jnp.bfloat16
    tb = 128

    B = x_nchw.shape[0]
    Bp = ((B + tb - 1) // tb) * tb
    if Bp != B:
        x_nchw = jnp.pad(x_nchw, ((0, Bp - B), (0, 0), (0, 0), (0, 0)))
    G = Bp // tb

    x5 = jnp.transpose(x_nchw, (0, 2, 3, 1)).reshape(Bp, 32, 96)

    body = functools.partial(_lenet_body, tb=tb)
    out = pl.pallas_call(
        body,
        out_shape=jax.ShapeDtypeStruct((Bp, 128), f32),
        grid=(G,),
        in_specs=[
            pl.BlockSpec((tb, 32, 96), lambda i: (i, 0, 0)),
            pl.BlockSpec((3, 96, 180), lambda i: (0, 0, 0)),
            pl.BlockSpec((1, 180), lambda i: (0, 0)),
            pl.BlockSpec((3, 174, 208), lambda i: (0, 0, 0)),
            pl.BlockSpec((1, 208), lambda i: (0, 0)),
            pl.BlockSpec((6, 192, 120), lambda i: (0, 0, 0)),
            pl.BlockSpec((1, 120), lambda i: (0, 0)),
            pl.BlockSpec((120, 84), lambda i: (0, 0)),
            pl.BlockSpec((1, 84), lambda i: (0, 0)),
            pl.BlockSpec((84, 128), lambda i: (0, 0)),
            pl.BlockSpec((1, 128), lambda i: (0, 0)),
        ],
        out_specs=pl.BlockSpec((tb, 128), lambda i: (i, 0)),
        compiler_params=pltpu.CompilerParams(
            dimension_semantics=("parallel",),
            vmem_limit_bytes=64 * 1024 * 1024),
    )(x5, wb1.astype(bf16), b1, wb2.astype(bf16), b2,
      wf1.astype(bf16), bf1, wf2.astype(bf16), bf2, wf3.astype(bf16), bf3)

    return out[:B, :10]


# speedup vs baseline: 5.7321x; 4.4591x over previous
import jax
import jax.numpy as jnp
from jax.experimental import pallas as pl
from jax.experimental.pallas import tpu as pltpu


def _lenet_body(x_ref, wb1_ref, b1_ref, wb2_ref, b2_ref,
                wf1_ref, bf1_ref, wf2_ref, bf2_ref,
                wf3_ref, bf3_ref, out_ref):
    f32 = jnp.float32
    bf16 = jnp.bfloat16
    bt = x_ref.shape[-1]

    xb = x_ref[...].astype(bf16)
    w1 = wb1_ref[...]
    bias1 = b1_ref[...][:, 0:1]

    y1 = []
    for h1 in range(30):
        sl = xb[:, h1:h1 + 3, :, :].reshape(288, bt)
        acc = jnp.dot(w1, sl, preferred_element_type=f32)
        y1.append(jnp.maximum(acc + bias1, 0.0).astype(bf16))

    w2 = wb2_ref[...]
    bias2 = b2_ref[...][:, 0:1]
    hp = []
    for h2 in range(15):
        m = jnp.maximum(y1[2 * h2], y1[2 * h2 + 1]).reshape(6, 2, 16, bt)
        hp.append(jnp.maximum(m[:, 0], m[:, 1]).reshape(96, bt))
    a2 = jnp.stack(hp)

    y2 = []
    for h3 in range(13):
        sl = a2[h3:h3 + 3].reshape(288, bt)
        acc = jnp.dot(w2, sl, preferred_element_type=f32)
        y2.append(jnp.maximum(acc + bias2, 0.0).astype(bf16))

    a4 = []
    for h4 in range(6):
        m = jnp.maximum(y2[2 * h4], y2[2 * h4 + 1]).reshape(16, 2, 8, bt)
        a4.append(jnp.maximum(m[:, 0], m[:, 1]).reshape(128, bt))
    a4 = jnp.stack(a4).reshape(768, bt)

    z1 = jnp.maximum(jnp.dot(wf1_ref[...], a4, preferred_element_type=f32)
                     + bf1_ref[...][:, 0:1], 0.0).astype(bf16)
    z2 = jnp.maximum(jnp.dot(wf2_ref[...], z1, preferred_element_type=f32)
                     + bf2_ref[...][:, 0:1], 0.0).astype(bf16)
    z3 = (jnp.dot(wf3_ref[...], z2, preferred_element_type=f32)
          + bf3_ref[...][:, 0:1])
    out_ref[...] = z3.astype(out_ref.dtype)


def _prep(wb1, b1, wb2, b2, wf1, bf1, wf2, bf2, wf3, bf3):
    f32 = jnp.float32
    t1 = wb1[:, :9, :6].reshape(3, 3, 3, 6)
    W1 = jnp.transpose(t1, (3, 2, 0, 1))
    t2 = wb2[:, :36, :16].reshape(3, 3, 12, 16)[:, :, :6, :]
    W2 = jnp.transpose(t2, (3, 2, 0, 1))

    E32 = jnp.stack([jnp.eye(32, 32, k=dx, dtype=f32) for dx in range(3)])
    WB1 = jnp.einsum('ocyd,dvw->ovcyw', W1, E32).reshape(192, 288)
    E16 = jnp.stack([jnp.eye(16, 16, k=dx, dtype=f32) for dx in range(3)])
    WB2 = jnp.einsum('ocyd,dvw->ovycw', W2, E16).reshape(256, 288)

    r = jnp.arange(192)
    WB1 = WB1[(r // 32) * 32 + 2 * (r % 16) + (r % 32) // 16]
    r = jnp.arange(256)
    WB2 = WB2[(r // 16) * 16 + 2 * (r % 8) + (r % 16) // 8]

    g = jnp.transpose(wf1.reshape(6, 6, 32, 120)[:, :, :16, :], (3, 0, 2, 1))
    F1 = jnp.zeros((120, 6, 16, 8), f32).at[:, :, :, :6].set(g).reshape(120, 768)
    F2 = wf2.T
    F3 = jnp.zeros((16, 84), f32).at[:10].set(wf3[:, :10].T)

    def rows(v):
        return jnp.tile(v.astype(f32)[:, None], (1, 128))

    return dict(
        WB1=WB1, WB2=WB2, F1=F1, F2=F2, F3=F3,
        B1=rows(jnp.repeat(b1[0, :6], 32)),
        B2=rows(jnp.repeat(b2[0, :16], 16)),
        BF1=rows(bf1[0]), BF2=rows(bf2[0]),
        BF3=rows(jnp.concatenate([bf3[0, :10], jnp.zeros(6, f32)])))


def kernel(x_nchw, wb1, b1, wb2, b2, wf1, bf1, wf2, bf2, wf3, bf3):
    f32 = jnp.float32
    bf16 = jnp.bfloat16
    bt = 512

    B = x_nchw.shape[0]
    Bp = ((B + bt - 1) // bt) * bt
    if Bp != B:
        x_nchw = jnp.pad(x_nchw, ((0, Bp - B), (0, 0), (0, 0), (0, 0)))
    G = Bp // bt

    xt = jnp.transpose(x_nchw, (1, 2, 3, 0))
    p = _prep(wb1, b1, wb2, b2, wf1, bf1, wf2, bf2, wf3, bf3)

    out = pl.pallas_call(
        _lenet_body,
        out_shape=jax.ShapeDtypeStruct((16, Bp), f32),
        grid=(G,),
        in_specs=[
            pl.BlockSpec((3, 32, 32, bt), lambda i: (0, 0, 0, i)),
            pl.BlockSpec((192, 288), lambda i: (0, 0)),
            pl.BlockSpec((192, 128), lambda i: (0, 0)),
            pl.BlockSpec((256, 288), lambda i: (0, 0)),
            pl.BlockSpec((256, 128), lambda i: (0, 0)),
            pl.BlockSpec((120, 768), lambda i: (0, 0)),
            pl.BlockSpec((120, 128), lambda i: (0, 0)),
            pl.BlockSpec((84, 120), lambda i: (0, 0)),
            pl.BlockSpec((84, 128), lambda i: (0, 0)),
            pl.BlockSpec((16, 84), lambda i: (0, 0)),
            pl.BlockSpec((16, 128), lambda i: (0, 0)),
        ],
        out_specs=pl.BlockSpec((16, bt), lambda i: (0, i)),
        compiler_params=pltpu.CompilerParams(
            dimension_semantics=("parallel",),
            vmem_limit_bytes=64 * 1024 * 1024),
    )(xt, p["WB1"].astype(bf16), p["B1"], p["WB2"].astype(bf16), p["B2"],
      p["F1"].astype(bf16), p["BF1"], p["F2"].astype(bf16), p["BF2"],
      p["F3"].astype(bf16), p["BF3"])

    return jnp.transpose(out[:10, :B], (1, 0))


# speedup vs baseline: 5.9351x; 1.0354x over previous
import jax
import jax.numpy as jnp
from jax.experimental import pallas as pl
from jax.experimental.pallas import tpu as pltpu


def _lenet_body(x_ref, wb1_ref, wb2_ref, wf1_ref, wf2_ref, wf3_ref,
                bias_ref, out_ref):
    f32 = jnp.float32
    bf16 = jnp.bfloat16
    bt = x_ref.shape[-1]

    xb = x_ref[...].astype(bf16)
    w1 = wb1_ref[...]
    bias = bias_ref[...]
    bias1 = bias[0:192, 0:1]

    y1 = []
    for h1 in range(30):
        sl = xb[:, h1:h1 + 3, :, :].reshape(288, bt)
        acc = jnp.dot(w1, sl, preferred_element_type=f32)
        y1.append(jnp.maximum(acc + bias1, 0.0).astype(bf16))

    w2 = wb2_ref[...]
    bias2 = bias[192:448, 0:1]
    hp = []
    for h2 in range(15):
        m = jnp.maximum(y1[2 * h2], y1[2 * h2 + 1]).reshape(6, 2, 16, bt)
        hp.append(jnp.maximum(m[:, 0], m[:, 1]).reshape(96, bt))
    a2 = jnp.stack(hp)

    y2 = []
    for h3 in range(13):
        sl = a2[h3:h3 + 3].reshape(288, bt)
        acc = jnp.dot(w2, sl, preferred_element_type=f32)
        y2.append(jnp.maximum(acc + bias2, 0.0).astype(bf16))

    a4 = []
    for h4 in range(6):
        m = jnp.maximum(y2[2 * h4], y2[2 * h4 + 1]).reshape(16, 2, 8, bt)
        a4.append(jnp.maximum(m[:, 0], m[:, 1]).reshape(128, bt))
    a4 = jnp.stack(a4).reshape(768, bt)

    z1 = jnp.maximum(jnp.dot(wf1_ref[...], a4, preferred_element_type=f32)
                     + bias[448:568, 0:1], 0.0).astype(bf16)
    z2 = jnp.maximum(jnp.dot(wf2_ref[...], z1, preferred_element_type=f32)
                     + bias[568:652, 0:1], 0.0).astype(bf16)
    z3 = (jnp.dot(wf3_ref[...], z2, preferred_element_type=f32)
          + bias[656:672, 0:1])
    out_ref[...] = jnp.transpose(z3, (1, 0)).astype(out_ref.dtype)


def _prep(wb1, b1, wb2, b2, wf1, bf1, wf2, bf2, wf3, bf3):
    f32 = jnp.float32
    t1 = wb1[:, :9, :6].reshape(3, 3, 3, 6)
    W1 = jnp.transpose(t1, (3, 2, 0, 1))
    t2 = wb2[:, :36, :16].reshape(3, 3, 12, 16)[:, :, :6, :]
    W2 = jnp.transpose(t2, (3, 2, 0, 1))

    E32 = jnp.stack([jnp.eye(32, 32, k=dx, dtype=f32) for dx in range(3)])
    WB1 = jnp.einsum('ocyd,dvw->ovcyw', W1, E32).reshape(192, 288)
    E16 = jnp.stack([jnp.eye(16, 16, k=dx, dtype=f32) for dx in range(3)])
    WB2 = jnp.einsum('ocyd,dvw->ovycw', W2, E16).reshape(256, 288)

    WB1 = jnp.transpose(WB1.reshape(6, 16, 2, 288), (0, 2, 1, 3)).reshape(192, 288)
    WB2 = jnp.transpose(WB2.reshape(16, 8, 2, 288), (0, 2, 1, 3)).reshape(256, 288)

    g = jnp.transpose(wf1.reshape(6, 6, 32, 120)[:, :, :16, :], (3, 0, 2, 1))
    F1 = jnp.zeros((120, 6, 16, 8), f32).at[:, :, :, :6].set(g).reshape(120, 768)
    F2 = wf2.T
    F3 = jnp.zeros((16, 84), f32).at[:10].set(wf3[:, :10].T)

    ball = jnp.concatenate([
        jnp.repeat(b1[0, :6], 32), jnp.repeat(b2[0, :16], 16),
        bf1[0], bf2[0], jnp.zeros(4, f32),
        bf3[0, :10], jnp.zeros(6, f32)]).astype(f32)
    biases = jnp.tile(ball[:, None], (1, 128))

    return dict(WB1=WB1, WB2=WB2, F1=F1, F2=F2, F3=F3, BIAS=biases)


def kernel(x_nchw, wb1, b1, wb2, b2, wf1, bf1, wf2, bf2, wf3, bf3):
    f32 = jnp.float32
    bf16 = jnp.bfloat16
    bt = 512

    B = x_nchw.shape[0]
    Bp = ((B + bt - 1) // bt) * bt
    if Bp != B:
        x_nchw = jnp.pad(x_nchw, ((0, Bp - B), (0, 0), (0, 0), (0, 0)))
    G = Bp // bt

    xt = jnp.transpose(x_nchw, (1, 2, 3, 0))
    p = _prep(wb1, b1, wb2, b2, wf1, bf1, wf2, bf2, wf3, bf3)

    out = pl.pallas_call(
        _lenet_body,
        out_shape=jax.ShapeDtypeStruct((Bp, 16), f32),
        grid=(G,),
        in_specs=[
            pl.BlockSpec((3, 32, 32, bt), lambda i: (0, 0, 0, i)),
            pl.BlockSpec((192, 288), lambda i: (0, 0)),
            pl.BlockSpec((256, 288), lambda i: (0, 0)),
            pl.BlockSpec((120, 768), lambda i: (0, 0)),
            pl.BlockSpec((84, 120), lambda i: (0, 0)),
            pl.BlockSpec((16, 84), lambda i: (0, 0)),
            pl.BlockSpec((672, 128), lambda i: (0, 0)),
        ],
        out_specs=pl.BlockSpec((bt, 16), lambda i: (i, 0)),
        compiler_params=pltpu.CompilerParams(
            dimension_semantics=("parallel",),
            vmem_limit_bytes=64 * 1024 * 1024),
    )(xt, p["WB1"].astype(bf16), p["WB2"].astype(bf16),
      p["F1"].astype(bf16), p["F2"].astype(bf16), p["F3"].astype(bf16),
      p["BIAS"])

    return out[:B, :10]


# speedup vs baseline: 6.0549x; 1.0202x over previous
import jax
import jax.numpy as jnp
from jax.experimental import pallas as pl
from jax.experimental.pallas import tpu as pltpu


def _lenet_body(x_ref, wb1_ref, wb2_ref, wf1_ref, wf2_ref, wf3_ref,
                bias_ref, out_ref):
    f32 = jnp.float32
    bf16 = jnp.bfloat16
    bt = x_ref.shape[-1]

    xb = x_ref[...].astype(bf16)
    w1 = wb1_ref[...]
    bias = bias_ref[...]
    bias1 = bias[0:192, 0:1]

    y1 = []
    for h1 in range(30):
        sl = xb[:, h1:h1 + 3, :, :].reshape(288, bt)
        acc = jnp.dot(w1, sl, preferred_element_type=f32)
        y1.append(jnp.maximum(acc + bias1, 0.0).astype(bf16))

    w2 = wb2_ref[...]
    bias2 = bias[192:448, 0:1]
    hp = []
    for h2 in range(15):
        m = jnp.maximum(y1[2 * h2], y1[2 * h2 + 1]).reshape(6, 2, 16, bt)
        hp.append(jnp.maximum(m[:, 0], m[:, 1]).reshape(96, bt))
    a2 = jnp.stack(hp)

    y2 = []
    for h3 in range(13):
        sl = a2[h3:h3 + 3].reshape(288, bt)
        acc = jnp.dot(w2, sl, preferred_element_type=f32)
        y2.append(jnp.maximum(acc + bias2, 0.0).astype(bf16))

    a4 = []
    for h4 in range(6):
        m = jnp.maximum(y2[2 * h4], y2[2 * h4 + 1]).reshape(16, 2, 8, bt)
        a4.append(jnp.maximum(m[:, 0], m[:, 1]).reshape(128, bt))
    a4 = jnp.stack(a4).reshape(768, bt)

    z1 = jnp.maximum(jnp.dot(wf1_ref[...], a4, preferred_element_type=f32)
                     + bias[448:568, 0:1], 0.0).astype(bf16)
    z2 = jnp.maximum(jnp.dot(wf2_ref[...], z1, preferred_element_type=f32)
                     + bias[568:652, 0:1], 0.0).astype(bf16)
    z3 = (jnp.dot(wf3_ref[...], z2, preferred_element_type=f32)
          + bias[656:672, 0:1])
    out_ref[...] = jnp.transpose(z3, (1, 0)).astype(out_ref.dtype)


def _prep(wb1, b1, wb2, b2, wf1, bf1, wf2, bf2, wf3, bf3):
    f32 = jnp.float32
    t1 = wb1[:, :9, :6].reshape(3, 3, 3, 6)
    W1 = jnp.transpose(t1, (3, 2, 0, 1))
    t2 = wb2[:, :36, :16].reshape(3, 3, 12, 16)[:, :, :6, :]
    W2 = jnp.transpose(t2, (3, 2, 0, 1))

    E32 = jnp.stack([jnp.eye(32, 32, k=dx, dtype=f32) for dx in range(3)])
    WB1 = jnp.einsum('ocyd,dvw->ovcyw', W1, E32).reshape(192, 288)
    E16 = jnp.stack([jnp.eye(16, 16, k=dx, dtype=f32) for dx in range(3)])
    WB2 = jnp.einsum('ocyd,dvw->ovycw', W2, E16).reshape(256, 288)

    WB1 = jnp.transpose(WB1.reshape(6, 16, 2, 288), (0, 2, 1, 3)).reshape(192, 288)
    WB2 = jnp.transpose(WB2.reshape(16, 8, 2, 288), (0, 2, 1, 3)).reshape(256, 288)

    g = jnp.transpose(wf1.reshape(6, 6, 32, 120)[:, :, :16, :], (3, 0, 2, 1))
    F1 = jnp.zeros((120, 6, 16, 8), f32).at[:, :, :, :6].set(g).reshape(120, 768)
    F2 = wf2.T
    F3 = jnp.zeros((16, 84), f32).at[:10].set(wf3[:, :10].T)

    ball = jnp.concatenate([
        jnp.repeat(b1[0, :6], 32), jnp.repeat(b2[0, :16], 16),
        bf1[0], bf2[0], jnp.zeros(4, f32),
        bf3[0, :10], jnp.zeros(6, f32)]).astype(f32)
    biases = jnp.tile(ball[:, None], (1, 128))

    return dict(WB1=WB1, WB2=WB2, F1=F1, F2=F2, F3=F3, BIAS=biases)


def kernel(x_nchw, wb1, b1, wb2, b2, wf1, bf1, wf2, bf2, wf3, bf3):
    f32 = jnp.float32
    bf16 = jnp.bfloat16
    bt = 1024

    B = x_nchw.shape[0]
    Bp = ((B + bt - 1) // bt) * bt
    if Bp != B:
        x_nchw = jnp.pad(x_nchw, ((0, Bp - B), (0, 0), (0, 0), (0, 0)))
    G = Bp // bt

    xt = jnp.transpose(x_nchw, (1, 2, 3, 0))
    p = _prep(wb1, b1, wb2, b2, wf1, bf1, wf2, bf2, wf3, bf3)

    out = pl.pallas_call(
        _lenet_body,
        out_shape=jax.ShapeDtypeStruct((Bp, 16), f32),
        grid=(G,),
        in_specs=[
            pl.BlockSpec((3, 32, 32, bt), lambda i: (0, 0, 0, i)),
            pl.BlockSpec((192, 288), lambda i: (0, 0)),
            pl.BlockSpec((256, 288), lambda i: (0, 0)),
            pl.BlockSpec((120, 768), lambda i: (0, 0)),
            pl.BlockSpec((84, 120), lambda i: (0, 0)),
            pl.BlockSpec((16, 84), lambda i: (0, 0)),
            pl.BlockSpec((672, 128), lambda i: (0, 0)),
        ],
        out_specs=pl.BlockSpec((bt, 16), lambda i: (i, 0)),
        compiler_params=pltpu.CompilerParams(
            dimension_semantics=("parallel",),
            vmem_limit_bytes=64 * 1024 * 1024),
    )(xt, p["WB1"].astype(bf16), p["WB2"].astype(bf16),
      p["F1"].astype(bf16), p["F2"].astype(bf16), p["F3"].astype(bf16),
      p["BIAS"])

    return out[:B, :10]


# speedup vs baseline: 6.0642x; 1.0015x over previous
import jax
import jax.numpy as jnp
from jax.experimental import pallas as pl
from jax.experimental.pallas import tpu as pltpu


def _lenet_body(x_ref, wb1_ref, wb2_ref, wf1_ref, wf2_ref, wf3_ref,
                bias_ref, out_ref):
    f32 = jnp.float32
    bf16 = jnp.bfloat16
    bt = x_ref.shape[-1]

    xb = x_ref[...].astype(bf16)
    w1 = wb1_ref[...]
    bias = bias_ref[...]
    bias1 = bias[0:192, 0:1]

    y1 = []
    for h1 in range(30):
        sl = xb[:, h1:h1 + 3, :, :].reshape(288, bt)
        acc = jnp.dot(w1, sl, preferred_element_type=f32)
        y1.append(jnp.maximum(acc + bias1, 0.0).astype(bf16))

    w2 = wb2_ref[...]
    bias2 = bias[192:448, 0:1]
    hp = []
    for h2 in range(15):
        m = jnp.maximum(y1[2 * h2], y1[2 * h2 + 1]).reshape(6, 2, 16, bt)
        hp.append(jnp.maximum(m[:, 0], m[:, 1]).reshape(96, bt))
    a2 = jnp.stack(hp)

    y2 = []
    for h3 in range(13):
        sl = a2[h3:h3 + 3].reshape(288, bt)
        acc = jnp.dot(w2, sl, preferred_element_type=f32)
        y2.append(jnp.maximum(acc + bias2, 0.0).astype(bf16))

    a4 = []
    for h4 in range(6):
        m = jnp.maximum(y2[2 * h4], y2[2 * h4 + 1]).reshape(16, 2, 8, bt)
        a4.append(jnp.maximum(m[:, 0], m[:, 1]).reshape(128, bt))
    a4 = jnp.stack(a4).reshape(768, bt)

    z1 = jnp.maximum(jnp.dot(wf1_ref[...], a4, preferred_element_type=f32)
                     + bias[448:568, 0:1], 0.0).astype(bf16)
    z2 = jnp.maximum(jnp.dot(wf2_ref[...], z1, preferred_element_type=f32)
                     + bias[568:652, 0:1], 0.0).astype(bf16)
    z3 = (jnp.dot(wf3_ref[...], z2, preferred_element_type=f32)
          + bias[656:672, 0:1])
    out_ref[...] = jnp.transpose(z3, (1, 0)).astype(out_ref.dtype)


def _prep(wb1, b1, wb2, b2, wf1, bf1, wf2, bf2, wf3, bf3):
    f32 = jnp.float32
    t1 = wb1[:, :9, :6].reshape(3, 3, 3, 6)
    W1 = jnp.transpose(t1, (3, 2, 0, 1))
    t2 = wb2[:, :36, :16].reshape(3, 3, 12, 16)[:, :, :6, :]
    W2 = jnp.transpose(t2, (3, 2, 0, 1))

    E32 = jnp.stack([jnp.eye(32, 32, k=dx, dtype=f32) for dx in range(3)])
    WB1 = jnp.einsum('ocyd,dvw->ovcyw', W1, E32).reshape(192, 288)
    E16 = jnp.stack([jnp.eye(16, 16, k=dx, dtype=f32) for dx in range(3)])
    WB2 = jnp.einsum('ocyd,dvw->ovycw', W2, E16).reshape(256, 288)

    WB1 = jnp.transpose(WB1.reshape(6, 16, 2, 288), (0, 2, 1, 3)).reshape(192, 288)
    WB2 = jnp.transpose(WB2.reshape(16, 8, 2, 288), (0, 2, 1, 3)).reshape(256, 288)

    g = jnp.transpose(wf1.reshape(6, 6, 32, 120)[:, :, :16, :], (3, 0, 2, 1))
    F1 = jnp.zeros((120, 6, 16, 8), f32).at[:, :, :, :6].set(g).reshape(120, 768)
    F2 = wf2.T
    F3 = jnp.zeros((16, 84), f32).at[:10].set(wf3[:, :10].T)

    ball = jnp.concatenate([
        jnp.repeat(b1[0, :6], 32), jnp.repeat(b2[0, :16], 16),
        bf1[0], bf2[0], jnp.zeros(4, f32),
        bf3[0, :10], jnp.zeros(6, f32)]).astype(f32)
    biases = jnp.tile(ball[:, None], (1, 128))

    return dict(WB1=WB1, WB2=WB2, F1=F1, F2=F2, F3=F3, BIAS=biases)


def kernel(x_nchw, wb1, b1, wb2, b2, wf1, bf1, wf2, bf2, wf3, bf3):
    f32 = jnp.float32
    bf16 = jnp.bfloat16
    bt = 1024

    B = x_nchw.shape[0]
    Bp = ((B + bt - 1) // bt) * bt
    if Bp != B:
        x_nchw = jnp.pad(x_nchw, ((0, Bp - B), (0, 0), (0, 0), (0, 0)))
    G = Bp // bt

    xt = jnp.transpose(x_nchw, (1, 2, 3, 0))
    p = _prep(wb1, b1, wb2, b2, wf1, bf1, wf2, bf2, wf3, bf3)

    out = pl.pallas_call(
        _lenet_body,
        out_shape=jax.ShapeDtypeStruct((Bp, 16), f32),
        grid=(G,),
        in_specs=[
            pl.BlockSpec((3, 32, 32, bt), lambda i: (0, 0, 0, i)),
            pl.BlockSpec((192, 288), lambda i: (0, 0)),
            pl.BlockSpec((256, 288), lambda i: (0, 0)),
            pl.BlockSpec((120, 768), lambda i: (0, 0)),
            pl.BlockSpec((84, 120), lambda i: (0, 0)),
            pl.BlockSpec((16, 84), lambda i: (0, 0)),
            pl.BlockSpec((672, 128), lambda i: (0, 0)),
        ],
        out_specs=pl.BlockSpec((bt, 16), lambda i: (i, 0)),
        compiler_params=pltpu.CompilerParams(
            dimension_semantics=("arbitrary",),
            vmem_limit_bytes=64 * 1024 * 1024),
    )(xt, p["WB1"].astype(bf16), p["WB2"].astype(bf16),
      p["F1"].astype(bf16), p["F2"].astype(bf16), p["F3"].astype(bf16),
      p["BIAS"])

    return out[:B, :10]
